# baseline (speedup 1.0000x reference)
import functools

import jax
import jax.numpy as jnp
from jax.experimental import pallas as pl
from jax.experimental.pallas import tpu as pltpu


def _pick_block(m, candidates):
    for c in candidates:
        if m % c == 0:
            return c
    return m


def _leaky(v):
    return jnp.where(v >= 0.0, v, 0.01 * v)


def _layer1_kernel(act_ref, x_ref, w1_ref, adj_ref, w2_ref,
                   s2_ref, adjc_ref, s1_scr):
    @pl.when(pl.program_id(0) == 0)
    def _():
        s = jnp.dot(x_ref[...], w1_ref[...],
                    preferred_element_type=jnp.float32)
        s = jnp.where(act_ref[0, 0] != 0, _leaky(s), s)
        s1_scr[...] = s.astype(s1_scr.dtype)

    adj_blk = adj_ref[...]
    adjc_ref[...] = adj_blk.astype(adjc_ref.dtype)
    z = jnp.dot(adj_blk.astype(jnp.bfloat16), s1_scr[...],
                preferred_element_type=jnp.float32)
    z = jnp.dot(z, w2_ref[...], preferred_element_type=jnp.float32)
    z = jnp.where(act_ref[0, 0] != 0, _leaky(z), z)
    s2_ref[...] = z.astype(s2_ref.dtype)


def _quant_split(s):
    mh = jnp.max(jnp.abs(s), axis=0, keepdims=True)
    kh = jnp.ceil(jnp.log2(jnp.maximum(mh, 1e-30))) - 8.0
    sch = jnp.exp2(kh)
    qh = (s * jnp.exp2(-kh)).astype(jnp.float8_e4m3fn)
    r = s - qh.astype(jnp.float32) * sch
    ml = jnp.max(jnp.abs(r), axis=0, keepdims=True)
    kl = jnp.ceil(jnp.log2(jnp.maximum(ml, 1e-30))) - 8.0
    scl = jnp.exp2(kl)
    ql = (r * jnp.exp2(-kl)).astype(jnp.float8_e4m3fn)
    return (jnp.concatenate([qh, ql], axis=1),
            jnp.concatenate([sch, scl], axis=1))


def _layer23_kernel(s_ref, w_ref, adj_ref, o_ref, q2_scr, sc2_scr,
                    s3_scr, q3_scr, sc3_scr, *, g, bm):
    i = pl.program_id(0)

    @pl.when(i == 0)
    def _():
        t = jnp.dot(s_ref[...].astype(jnp.float32), w_ref[...],
                    preferred_element_type=jnp.float32)
        q, sc = _quant_split(t)
        q2_scr[...] = q
        sc2_scr[...] = sc

    @pl.when(i == g)
    def _():
        q, sc = _quant_split(s3_scr[...].astype(jnp.float32))
        q3_scr[...] = q
        sc3_scr[...] = sc

    @pl.when(i < g)
    def _():
        z = jax.lax.dot_general(
            adj_ref[...], q2_scr[...], (((1,), (0,)), ((), ())),
            preferred_element_type=jnp.float32,
        )
        z = z * sc2_scr[...]
        n = z.shape[1] // 2
        s3_scr[pl.ds(i * bm, bm), :] = (
            (z[:, :n] + z[:, n:]).astype(s3_scr.dtype))

    @pl.when(i >= g)
    def _():
        z = jax.lax.dot_general(
            adj_ref[...], q3_scr[...], (((1,), (0,)), ((), ())),
            preferred_element_type=jnp.float32,
        )
        z = z * sc3_scr[...]
        n = z.shape[1] // 2
        o_ref[...] = z[:, :n] + z[:, n:]


def _layer1(adj, x, w1t, w2t, act):
    m, k = adj.shape
    n1 = w1t.shape[1]
    n2 = w2t.shape[1]
    bm = _pick_block(m, (200, 80, 16, 8))
    return pl.pallas_call(
        _layer1_kernel,
        grid=(m // bm,),
        in_specs=[
            pl.BlockSpec(memory_space=pltpu.SMEM),
            pl.BlockSpec((m, x.shape[1]), lambda i: (0, 0)),
            pl.BlockSpec((x.shape[1], n1), lambda i: (0, 0)),
            pl.BlockSpec((bm, k), lambda i: (i, 0)),
            pl.BlockSpec((n1, n2), lambda i: (0, 0)),
        ],
        out_specs=[
            pl.BlockSpec((bm, n2), lambda i: (i, 0)),
            pl.BlockSpec((bm, k), lambda i: (i, 0)),
        ],
        out_shape=[
            jax.ShapeDtypeStruct((m, n2), jnp.bfloat16),
            jax.ShapeDtypeStruct((m, k), jnp.float8_e4m3fn),
        ],
        scratch_shapes=[pltpu.VMEM((m, n1), jnp.bfloat16)],
    )(act, x, w1t, adj, w2t)


def _layer23(adj_c, s, wt):
    m, k = adj_c.shape
    n = s.shape[1]
    n2 = wt.shape[1]
    bm = _pick_block(m, (400, 200, 80, 16, 8))
    g = m // bm
    return pl.pallas_call(
        functools.partial(_layer23_kernel, g=g, bm=bm),
        grid=(2 * g,),
        in_specs=[
            pl.BlockSpec((m, n), lambda i: (0, 0)),
            pl.BlockSpec((n, n2), lambda i: (0, 0)),
            pl.BlockSpec((bm, k), lambda i: (i % g, 0)),
        ],
        out_specs=pl.BlockSpec(
            (bm, n2), lambda i: (jnp.maximum(i - g, 0), 0)),
        out_shape=jax.ShapeDtypeStruct((m, n2), jnp.float32),
        scratch_shapes=[
            pltpu.VMEM((m, 2 * n2), jnp.float8_e4m3fn),
            pltpu.VMEM((1, 2 * n2), jnp.float32),
            pltpu.VMEM((m, n2), jnp.bfloat16),
            pltpu.VMEM((m, 2 * n2), jnp.float8_e4m3fn),
            pltpu.VMEM((1, 2 * n2), jnp.float32),
        ],
    )(s, wt, adj_c)


def _gram_kernel(a_ref, bt_ref, o_ref):
    a = a_ref[...].astype(jnp.bfloat16)
    bt = bt_ref[...].astype(jnp.bfloat16)
    g = jnp.dot(a, bt, preferred_element_type=jnp.float32)
    o_ref[...] = jax.nn.sigmoid(g)


def _gram_sigmoid(z, zt):
    m = z.shape[0]
    k = z.shape[1]
    bm = _pick_block(m, (400, 200, 80, 16, 8))
    return pl.pallas_call(
        _gram_kernel,
        grid=(m // bm,),
        in_specs=[
            pl.BlockSpec((bm, k), lambda i: (i, 0)),
            pl.BlockSpec((k, m), lambda i: (0, 0)),
        ],
        out_specs=pl.BlockSpec((bm, m), lambda i: (i, 0)),
        out_shape=jax.ShapeDtypeStruct((m, m), jnp.float32),
    )(z, zt)


def kernel(x, adj, active, W1, W2, W3):
    act = jnp.asarray(active, jnp.int32).reshape(1, 1)
    s2, adj_c = _layer1(adj, x, W1.T, W2.T, act)
    z3 = _layer23(adj_c, s2, W3.T)
    adj_hat = _gram_sigmoid(z3, z3.T)
    return (z3, adj_hat)


# speedup vs baseline: 1.0165x; 1.0165x over previous
import functools

import jax
import jax.numpy as jnp
from jax.experimental import pallas as pl
from jax.experimental.pallas import tpu as pltpu


def _pick_block(m, candidates):
    for c in candidates:
        if m % c == 0:
            return c
    return m


def _leaky(v):
    return jnp.where(v >= 0.0, v, 0.01 * v)


def _layer1_kernel(act_ref, x_ref, w1_ref, adj_ref, w2_ref,
                   s2_ref, adjc_ref, s1_scr):
    @pl.when(pl.program_id(0) == 0)
    def _():
        s = jnp.dot(x_ref[...], w1_ref[...],
                    preferred_element_type=jnp.float32)
        s = jnp.where(act_ref[0, 0] != 0, _leaky(s), s)
        s1_scr[...] = s.astype(s1_scr.dtype)

    adj_blk = adj_ref[...]
    adjc_ref[...] = adj_blk.astype(adjc_ref.dtype)
    z = jnp.dot(adj_blk.astype(jnp.bfloat16), s1_scr[...],
                preferred_element_type=jnp.float32)
    z = jnp.dot(z, w2_ref[...], preferred_element_type=jnp.float32)
    z = jnp.where(act_ref[0, 0] != 0, _leaky(z), z)
    s2_ref[...] = z.astype(s2_ref.dtype)


def _quant_split(s):
    mh = jnp.max(jnp.abs(s), axis=0, keepdims=True)
    kh = jnp.ceil(jnp.log2(jnp.maximum(mh, 1e-30))) - 8.0
    sch = jnp.exp2(kh)
    qh = (s * jnp.exp2(-kh)).astype(jnp.float8_e4m3fn)
    r = s - qh.astype(jnp.float32) * sch
    ml = jnp.max(jnp.abs(r), axis=0, keepdims=True)
    kl = jnp.ceil(jnp.log2(jnp.maximum(ml, 1e-30))) - 8.0
    scl = jnp.exp2(kl)
    ql = (r * jnp.exp2(-kl)).astype(jnp.float8_e4m3fn)
    return (jnp.concatenate([qh, ql], axis=1),
            jnp.concatenate([sch, scl], axis=1))


def _layer23_kernel(s_ref, w_ref, adj_ref, o_ref, q2_scr, sc2_scr,
                    s3_scr, q3_scr, sc3_scr, *, g, bm):
    i = pl.program_id(0)

    @pl.when(i == 0)
    def _():
        t = jnp.dot(s_ref[...].astype(jnp.float32), w_ref[...],
                    preferred_element_type=jnp.float32)
        q, sc = _quant_split(t)
        q2_scr[...] = q
        sc2_scr[...] = sc

    @pl.when(i == g)
    def _():
        q, sc = _quant_split(s3_scr[...].astype(jnp.float32))
        q3_scr[...] = q
        sc3_scr[...] = sc

    @pl.when(i < g)
    def _():
        z = jax.lax.dot_general(
            adj_ref[...], q2_scr[...], (((1,), (0,)), ((), ())),
            preferred_element_type=jnp.float32,
        )
        z = z * sc2_scr[...]
        n = z.shape[1] // 2
        s3_scr[pl.ds(i * bm, bm), :] = (
            (z[:, :n] + z[:, n:]).astype(s3_scr.dtype))

    @pl.when(i >= g)
    def _():
        z = jax.lax.dot_general(
            adj_ref[...], q3_scr[...], (((1,), (0,)), ((), ())),
            preferred_element_type=jnp.float32,
        )
        z = z * sc3_scr[...]
        n = z.shape[1] // 2
        o_ref[...] = z[:, :n] + z[:, n:]


def _layer1(adj, x, w1t, w2t, act):
    m, k = adj.shape
    n1 = w1t.shape[1]
    n2 = w2t.shape[1]
    bm = _pick_block(m, (400, 200, 80, 16, 8))
    return pl.pallas_call(
        _layer1_kernel,
        grid=(m // bm,),
        in_specs=[
            pl.BlockSpec(memory_space=pltpu.SMEM),
            pl.BlockSpec((m, x.shape[1]), lambda i: (0, 0)),
            pl.BlockSpec((x.shape[1], n1), lambda i: (0, 0)),
            pl.BlockSpec((bm, k), lambda i: (i, 0)),
            pl.BlockSpec((n1, n2), lambda i: (0, 0)),
        ],
        out_specs=[
            pl.BlockSpec((bm, n2), lambda i: (i, 0)),
            pl.BlockSpec((bm, k), lambda i: (i, 0)),
        ],
        out_shape=[
            jax.ShapeDtypeStruct((m, n2), jnp.bfloat16),
            jax.ShapeDtypeStruct((m, k), jnp.float8_e4m3fn),
        ],
        scratch_shapes=[pltpu.VMEM((m, n1), jnp.bfloat16)],
    )(act, x, w1t, adj, w2t)


def _layer23(adj_c, s, wt):
    m, k = adj_c.shape
    n = s.shape[1]
    n2 = wt.shape[1]
    bm = _pick_block(m, (400, 200, 80, 16, 8))
    g = m // bm
    return pl.pallas_call(
        functools.partial(_layer23_kernel, g=g, bm=bm),
        grid=(2 * g,),
        in_specs=[
            pl.BlockSpec((m, n), lambda i: (0, 0)),
            pl.BlockSpec((n, n2), lambda i: (0, 0)),
            pl.BlockSpec((bm, k), lambda i: (i % g, 0)),
        ],
        out_specs=pl.BlockSpec(
            (bm, n2), lambda i: (jnp.maximum(i - g, 0), 0)),
        out_shape=jax.ShapeDtypeStruct((m, n2), jnp.float32),
        scratch_shapes=[
            pltpu.VMEM((m, 2 * n2), jnp.float8_e4m3fn),
            pltpu.VMEM((1, 2 * n2), jnp.float32),
            pltpu.VMEM((m, n2), jnp.bfloat16),
            pltpu.VMEM((m, 2 * n2), jnp.float8_e4m3fn),
            pltpu.VMEM((1, 2 * n2), jnp.float32),
        ],
    )(s, wt, adj_c)


def _gram_kernel(a_ref, bt_ref, o_ref):
    a = a_ref[...].astype(jnp.bfloat16)
    bt = bt_ref[...].astype(jnp.bfloat16)
    g = jnp.dot(a, bt, preferred_element_type=jnp.float32)
    o_ref[...] = jax.nn.sigmoid(g)


def _gram_sigmoid(z, zt):
    m = z.shape[0]
    k = z.shape[1]
    bm = _pick_block(m, (400, 200, 80, 16, 8))
    return pl.pallas_call(
        _gram_kernel,
        grid=(m // bm,),
        in_specs=[
            pl.BlockSpec((bm, k), lambda i: (i, 0)),
            pl.BlockSpec((k, m), lambda i: (0, 0)),
        ],
        out_specs=pl.BlockSpec((bm, m), lambda i: (i, 0)),
        out_shape=jax.ShapeDtypeStruct((m, m), jnp.float32),
    )(z, zt)


def kernel(x, adj, active, W1, W2, W3):
    act = jnp.asarray(active, jnp.int32).reshape(1, 1)
    s2, adj_c = _layer1(adj, x, W1.T, W2.T, act)
    z3 = _layer23(adj_c, s2, W3.T)
    adj_hat = _gram_sigmoid(z3, z3.T)
    return (z3, adj_hat)
